# baseline (speedup 1.0000x reference)
import jax
import jax.numpy as jnp
from jax import lax
from jax.experimental import pallas as pl
from jax.experimental.pallas import tpu as pltpu
from jax.experimental.pallas import tpu_sc as plsc

B = 1024
N_SRC = 4096
HW = 64 * 64
ROW = 2 * HW
NC = 2
NS = 16
NW = NC * NS
SYS_PER = B // NW
SEG_WIN = 48
L = 16
BS = 4
NB = SYS_PER // BS


def _sc_body(grid_hbm, params_hbm, order_hbm, seg_hbm, out_hbm,
             seg_v, order_v, params_v,
             grid_a, grid_b, acc_a, acc_b,
             sem_ga, sem_gb, sem_oa, sem_ob):
    wid = lax.axis_index("s") * NC + lax.axis_index("c")
    base = wid * SYS_PER
    pltpu.sync_copy(order_hbm, order_v)
    pltpu.sync_copy(params_hbm, params_v)
    pltpu.sync_copy(seg_hbm.at[pl.ds(base, SEG_WIN)], seg_v)

    iota = lax.iota(jnp.int32, L)
    two_iota = iota * 2
    zeros = jnp.zeros((L,), jnp.float32)

    def grid_cp(t, gbuf, sem):
        return pltpu.make_async_copy(
            grid_hbm.at[pl.ds(base + t * BS, BS)], gbuf, sem)

    def out_cp(t, abuf, sem):
        return pltpu.make_async_copy(
            abuf, out_hbm.at[pl.ds(base + t * BS, BS)], sem)

    def load_params(s):
        sidv = plsc.load_gather(
            order_v, [jnp.broadcast_to(s, (L,)).astype(jnp.int32)]
        )
        p = sidv * 4
        x0 = plsc.load_gather(params_v, [p])
        y0 = plsc.load_gather(params_v, [p + 1])
        amp = plsc.load_gather(params_v, [p + 2])
        sg = plsc.load_gather(params_v, [p + 3])
        nk = -0.5 / (sg * sg)
        return x0, y0, amp, nk

    def compute_batch(t, gbuf, abuf):
        for j in range(BS):
            i = t * BS + j
            jv = jnp.full((L,), j, jnp.int32)
            svec = plsc.load_gather(seg_v, [(i + iota).astype(jnp.int32)])
            s0 = jnp.max(jnp.where(iota == 0, svec, 0))
            s1 = jnp.max(jnp.where(iota == 1, svec, 0))

            @pl.when(s1 == s0)
            def _empty():
                @plsc.parallel_loop(0, HW // L, unroll=8)
                def zero_chunk(k):
                    abuf[j, pl.ds(k * L, L)] = zeros

            @pl.when(s1 > s0)
            def _nonempty():
                x0, y0, amp, nk = load_params(s0)

                @plsc.parallel_loop(0, HW // L, unroll=8)
                def first_chunk(k):
                    ix = two_iota + k * 2 * L
                    gx = plsc.load_gather(gbuf, [jv, ix])
                    gy = plsc.load_gather(gbuf, [jv, ix + 1])
                    dx = gx - x0
                    dy = gy - y0
                    abuf[j, pl.ds(k * L, L)] = (
                        amp * jnp.exp((dx * dx + dy * dy) * nk)
                    )

                def do_source(s, c):
                    x0, y0, amp, nk = load_params(s)

                    @plsc.parallel_loop(0, HW // L, unroll=8)
                    def do_chunk(k):
                        ix = two_iota + k * 2 * L
                        gx = plsc.load_gather(gbuf, [jv, ix])
                        gy = plsc.load_gather(gbuf, [jv, ix + 1])
                        dx = gx - x0
                        dy = gy - y0
                        val = amp * jnp.exp((dx * dx + dy * dy) * nk)
                        plsc.addupdate(abuf.at[j, pl.ds(k * L, L)], val)
                    return c
                lax.fori_loop(s0 + 1, s1, do_source, 0)

    grid_cp(0, grid_a, sem_ga).start()

    def do_batch(t, c):
        even = lax.rem(t, 2) == 0

        @pl.when(even)
        def _a():
            grid_cp(t, grid_a, sem_ga).wait()

            @pl.when(t + 1 < NB)
            def _pf():
                grid_cp(t + 1, grid_b, sem_gb).start()

            @pl.when(t >= 2)
            def _wo():
                out_cp(t - 2, acc_a, sem_oa).wait()

            compute_batch(t, grid_a, acc_a)
            out_cp(t, acc_a, sem_oa).start()

        @pl.when(jnp.logical_not(even))
        def _b():
            grid_cp(t, grid_b, sem_gb).wait()

            @pl.when(t + 1 < NB)
            def _pf():
                grid_cp(t + 1, grid_a, sem_ga).start()

            @pl.when(t >= 2)
            def _wo():
                out_cp(t - 2, acc_b, sem_ob).wait()

            compute_batch(t, grid_b, acc_b)
            out_cp(t, acc_b, sem_ob).start()
        return c

    lax.fori_loop(0, NB, do_batch, 0)
    out_cp(NB - 2, acc_a, sem_oa).wait()
    out_cp(NB - 1, acc_b, sem_ob).wait()


def kernel(source_grid, blob_params, sys_idx):
    source_grid = source_grid.astype(jnp.float32)
    idx = sys_idx.astype(jnp.int32)
    order = jnp.argsort(idx).astype(jnp.int32)
    sorted_sys = jnp.sort(idx)
    seg = jnp.searchsorted(
        sorted_sys, jnp.arange(B + 1, dtype=jnp.int32), side="left"
    ).astype(jnp.int32)
    seg = jnp.concatenate(
        [seg, jnp.full((NW * SYS_PER + SEG_WIN - (B + 1),), N_SRC, jnp.int32)]
    )

    grid2 = source_grid.reshape(B, ROW)
    params_flat = blob_params.astype(jnp.float32).reshape(-1)

    mesh = plsc.VectorSubcoreMesh(core_axis_name="c", subcore_axis_name="s")
    run = pl.kernel(
        _sc_body,
        mesh=mesh,
        compiler_params=pltpu.CompilerParams(needs_layout_passes=False),
        out_type=jax.ShapeDtypeStruct((B, HW), jnp.float32),
        scratch_types=[
            pltpu.VMEM((SEG_WIN,), jnp.int32),
            pltpu.VMEM((N_SRC,), jnp.int32),
            pltpu.VMEM((4 * N_SRC,), jnp.float32),
            pltpu.VMEM((BS, ROW), jnp.float32),
            pltpu.VMEM((BS, ROW), jnp.float32),
            pltpu.VMEM((BS, HW), jnp.float32),
            pltpu.VMEM((BS, HW), jnp.float32),
            pltpu.SemaphoreType.DMA,
            pltpu.SemaphoreType.DMA,
            pltpu.SemaphoreType.DMA,
            pltpu.SemaphoreType.DMA,
        ],
    )
    out = run(grid2, params_flat, order, seg)
    return out.reshape(B, 64, 64)


# speedup vs baseline: 1.3321x; 1.3321x over previous
import jax
import jax.numpy as jnp
from jax import lax
from jax.experimental import pallas as pl
from jax.experimental.pallas import tpu as pltpu
from jax.experimental.pallas import tpu_sc as plsc

B = 1024
N_SRC = 4096
HW = 64 * 64
ROW = 2 * HW
NC = 2
NS = 16
NW = NC * NS
SYS_PER = B // NW
SEG_WIN = 48
L = 16
BS = 4
NB = SYS_PER // BS


def _sc_body(grid_hbm, params_hbm, order_hbm, seg_hbm, out_hbm,
             seg_v, order_v, params_v,
             grid_a, grid_b, acc_a, acc_b,
             sem_ga, sem_gb, sem_oa, sem_ob):
    wid = lax.axis_index("s") * NC + lax.axis_index("c")
    base = wid * SYS_PER
    pltpu.sync_copy(order_hbm, order_v)
    pltpu.sync_copy(params_hbm, params_v)
    pltpu.sync_copy(seg_hbm.at[pl.ds(base, SEG_WIN)], seg_v)

    iota = lax.iota(jnp.int32, L)
    two_iota = iota * 2
    zeros = jnp.zeros((L,), jnp.float32)

    def grid_cp(t, gbuf, sem):
        return pltpu.make_async_copy(
            grid_hbm.at[pl.ds(base + t * BS, BS)], gbuf, sem)

    def out_cp(t, abuf, sem):
        return pltpu.make_async_copy(
            abuf, out_hbm.at[pl.ds(base + t * BS, BS)], sem)

    def load_params(s):
        sidv = plsc.load_gather(
            order_v, [jnp.broadcast_to(s, (L,)).astype(jnp.int32)]
        )
        p = sidv * 4
        x0 = plsc.load_gather(params_v, [p])
        y0 = plsc.load_gather(params_v, [p + 1])
        amp = plsc.load_gather(params_v, [p + 2])
        sg = plsc.load_gather(params_v, [p + 3])
        nk = -0.5 / (sg * sg)
        return x0, y0, amp, nk

    def compute_batch(t, gbuf, abuf):
        for j in range(BS):
            i = t * BS + j
            jv = jnp.full((L,), j, jnp.int32)
            svec = plsc.load_gather(seg_v, [(i + iota).astype(jnp.int32)])
            s0 = jnp.max(jnp.where(iota == 0, svec, 0))
            s1 = jnp.max(jnp.where(iota == 1, svec, 0))

            @pl.when(s1 == s0)
            def _empty():
                @plsc.parallel_loop(0, HW // L, unroll=8)
                def zero_chunk(k):
                    abuf[j, pl.ds(k * L, L)] = zeros

            @pl.when(s1 > s0)
            def _nonempty():
                x0, y0, amp, nk = load_params(s0)

                @plsc.parallel_loop(0, HW // L, unroll=8)
                def first_chunk(k):
                    ix = two_iota + k * 2 * L
                    gx = plsc.load_gather(gbuf, [jv, ix])
                    gy = plsc.load_gather(gbuf, [jv, ix + 1])
                    dx = gx - x0
                    dy = gy - y0
                    abuf[j, pl.ds(k * L, L)] = (
                        amp * jnp.exp((dx * dx + dy * dy) * nk)
                    )

                def do_source(s, c):
                    x0, y0, amp, nk = load_params(s)

                    @plsc.parallel_loop(0, HW // L, unroll=8)
                    def do_chunk(k):
                        ix = two_iota + k * 2 * L
                        gx = plsc.load_gather(gbuf, [jv, ix])
                        gy = plsc.load_gather(gbuf, [jv, ix + 1])
                        dx = gx - x0
                        dy = gy - y0
                        val = amp * jnp.exp((dx * dx + dy * dy) * nk)
                        plsc.addupdate(abuf.at[j, pl.ds(k * L, L)], val)
                    return c
                lax.fori_loop(s0 + 1, s1, do_source, 0)

    grid_cp(0, grid_a, sem_ga).start()

    def do_batch(t, c):
        even = lax.rem(t, 2) == 0

        @pl.when(even)
        def _a():
            grid_cp(t, grid_a, sem_ga).wait()

            @pl.when(t + 1 < NB)
            def _pf():
                grid_cp(t + 1, grid_b, sem_gb).start()

            @pl.when(t >= 2)
            def _wo():
                out_cp(t - 2, acc_a, sem_oa).wait()

            compute_batch(t, grid_a, acc_a)
            out_cp(t, acc_a, sem_oa).start()

        @pl.when(jnp.logical_not(even))
        def _b():
            grid_cp(t, grid_b, sem_gb).wait()

            @pl.when(t + 1 < NB)
            def _pf():
                grid_cp(t + 1, grid_a, sem_ga).start()

            @pl.when(t >= 2)
            def _wo():
                out_cp(t - 2, acc_b, sem_ob).wait()

            compute_batch(t, grid_b, acc_b)
            out_cp(t, acc_b, sem_ob).start()
        return c

    lax.fori_loop(0, NB, do_batch, 0)
    out_cp(NB - 2, acc_a, sem_oa).wait()
    out_cp(NB - 1, acc_b, sem_ob).wait()


def kernel(source_grid, blob_params, sys_idx):
    source_grid = source_grid.astype(jnp.float32)
    idx = sys_idx.astype(jnp.int32)
    sorted_sys, order = lax.sort(
        (idx, jnp.arange(N_SRC, dtype=jnp.int32)), num_keys=1
    )
    seg = jnp.searchsorted(
        sorted_sys, jnp.arange(B + 1, dtype=jnp.int32), side="left",
        method="compare_all",
    ).astype(jnp.int32)
    seg = jnp.concatenate(
        [seg, jnp.full((NW * SYS_PER + SEG_WIN - (B + 1),), N_SRC, jnp.int32)]
    )

    grid2 = source_grid.reshape(B, ROW)
    params_flat = blob_params.astype(jnp.float32).reshape(-1)

    mesh = plsc.VectorSubcoreMesh(core_axis_name="c", subcore_axis_name="s")
    run = pl.kernel(
        _sc_body,
        mesh=mesh,
        compiler_params=pltpu.CompilerParams(needs_layout_passes=False),
        out_type=jax.ShapeDtypeStruct((B, HW), jnp.float32),
        scratch_types=[
            pltpu.VMEM((SEG_WIN,), jnp.int32),
            pltpu.VMEM((N_SRC,), jnp.int32),
            pltpu.VMEM((4 * N_SRC,), jnp.float32),
            pltpu.VMEM((BS, ROW), jnp.float32),
            pltpu.VMEM((BS, ROW), jnp.float32),
            pltpu.VMEM((BS, HW), jnp.float32),
            pltpu.VMEM((BS, HW), jnp.float32),
            pltpu.SemaphoreType.DMA,
            pltpu.SemaphoreType.DMA,
            pltpu.SemaphoreType.DMA,
            pltpu.SemaphoreType.DMA,
        ],
    )
    out = run(grid2, params_flat, order, seg)
    return out.reshape(B, 64, 64)
